# baseline (speedup 1.0000x reference)
import functools

import jax
import jax.numpy as jnp
from jax import lax
from jax.experimental import pallas as pl
from jax.experimental.pallas import tpu as pltpu
from jax.experimental.pallas import tpu_sc as plsc

_info = plsc.get_sparse_core_info()
_NC, _NS = _info.num_cores, _info.num_subcores
_NW = _NC * _NS


def _make_gather(batch: int, hist: int, emb_dim: int, nb: int):
    rows_per_w = batch // _NW
    n_chunks = rows_per_w // nb
    assert batch % _NW == 0 and rows_per_w % nb == 0
    n_flat = nb * hist
    assert (n_flat * emb_dim) % 8 == 0
    segs = list(range(0, hist - 15, 16))
    if hist % 16:
        segs.append(hist - 16)
    mesh = plsc.VectorSubcoreMesh(core_axis_name="c", subcore_axis_name="s")

    @functools.partial(
        pl.kernel,
        mesh=mesh,
        out_type=jax.ShapeDtypeStruct((batch * hist, emb_dim), jnp.float32),
        scratch_types=[
            pltpu.VMEM((nb, 128), jnp.int32),
            pltpu.VMEM((n_flat,), jnp.int32),
            pltpu.VMEM((n_flat, emb_dim), jnp.float32),
            pltpu.SemaphoreType.DMA,
        ],
        compiler_params=pltpu.CompilerParams(use_tc_tiling_on_sc=False),
    )
    def gather_kernel(idx_hbm, table_hbm, out_hbm, idx2_v, flat_v, rows_v, sem):
        wid = lax.axis_index("s") * _NC + lax.axis_index("c")
        base = wid * rows_per_w

        def body(i, carry):
            r0 = base + i * nb
            pltpu.sync_copy(idx_hbm.at[pl.ds(r0, nb), :], idx2_v)
            for r in range(nb):
                for c0 in segs:
                    flat_v[pl.ds(r * hist + c0, 16)] = idx2_v[r, pl.ds(c0, 16)]
            pltpu.async_copy(table_hbm.at[flat_v], rows_v, sem).wait()
            pltpu.sync_copy(rows_v, out_hbm.at[pl.ds(r0 * hist, n_flat)])
            return carry

        lax.fori_loop(0, n_chunks, body, 0)

    return gather_kernel


def kernel(inputs, table):
    batch, hist = inputs.shape
    n_vocab, emb_dim = table.shape
    idx_p = jnp.pad(inputs, ((0, 0), (0, 128 - hist)))
    flat = _make_gather(batch, hist, emb_dim, nb=16)(idx_p, table)
    return flat.reshape(batch, hist, emb_dim)


# speedup vs baseline: 1.6348x; 1.6348x over previous
import functools

import jax
import jax.numpy as jnp
from jax import lax
from jax.experimental import pallas as pl
from jax.experimental.pallas import tpu as pltpu
from jax.experimental.pallas import tpu_sc as plsc

_info = plsc.get_sparse_core_info()
_NC, _NS = _info.num_cores, _info.num_subcores
_NW = _NC * _NS


def _make_gather(batch: int, hist: int, emb_dim: int, nb: int):
    rows_per_w = batch // _NW
    n_chunks = rows_per_w // nb
    assert batch % _NW == 0 and rows_per_w % nb == 0
    n_flat = nb * hist
    segs = list(range(0, hist - 15, 16))
    if hist % 16:
        segs.append(hist - 16)
    mesh = plsc.VectorSubcoreMesh(core_axis_name="c", subcore_axis_name="s")

    @functools.partial(
        pl.kernel,
        mesh=mesh,
        out_type=jax.ShapeDtypeStruct((batch, hist, emb_dim), jnp.float32),
        scratch_types=[
            pltpu.VMEM((nb, hist), jnp.int32),
            pltpu.VMEM((n_flat,), jnp.int32),
            pltpu.VMEM((n_flat, emb_dim), jnp.float32),
            pltpu.SemaphoreType.DMA,
            pltpu.SemaphoreType.DMA,
        ],
        compiler_params=pltpu.CompilerParams(use_tc_tiling_on_sc=False),
    )
    def gather_kernel(idx_hbm, table_hbm, out_hbm, idx2_v, flat_v, rows_v,
                      sem, sem2):
        wid = lax.axis_index("s") * _NC + lax.axis_index("c")
        base = wid * rows_per_w

        def body(i, carry):
            r0 = base + i * nb
            pltpu.sync_copy(idx_hbm.at[pl.ds(r0, nb), :], idx2_v)
            for r in range(nb):
                for c0 in segs:
                    flat_v[pl.ds(r * hist + c0, 16)] = idx2_v[r, pl.ds(c0, 16)]
            pltpu.async_copy(table_hbm.at[flat_v], rows_v, sem).wait()
            handles = [
                pltpu.async_copy(
                    rows_v.at[pl.ds(r * hist, hist), :],
                    out_hbm.at[r0 + r],
                    sem2,
                )
                for r in range(nb)
            ]
            for h in handles:
                h.wait()
            return carry

        lax.fori_loop(0, n_chunks, body, 0)

    return gather_kernel


def kernel(inputs, table):
    batch, hist = inputs.shape
    n_vocab, emb_dim = table.shape
    return _make_gather(batch, hist, emb_dim, nb=32)(inputs, table)


# speedup vs baseline: 1.6618x; 1.0166x over previous
import functools

import jax
import jax.numpy as jnp
from jax import lax
from jax.experimental import pallas as pl
from jax.experimental.pallas import tpu as pltpu
from jax.experimental.pallas import tpu_sc as plsc

_info = plsc.get_sparse_core_info()
_NC, _NS = _info.num_cores, _info.num_subcores
_NW = _NC * _NS


def _make_gather(batch: int, hist: int, emb_dim: int, nb: int):
    rows_per_w = batch // _NW
    n_chunks = rows_per_w // nb
    assert batch % _NW == 0 and rows_per_w % nb == 0
    n_flat = nb * hist
    mesh = plsc.VectorSubcoreMesh(core_axis_name="c", subcore_axis_name="s")

    @functools.partial(
        pl.kernel,
        mesh=mesh,
        out_type=jax.ShapeDtypeStruct((batch, hist, emb_dim), jnp.float32),
        scratch_types=[
            pltpu.VMEM((n_flat,), jnp.int32),
            pltpu.VMEM((n_flat, emb_dim), jnp.float32),
            pltpu.SemaphoreType.DMA,
            pltpu.SemaphoreType.DMA,
        ],
        compiler_params=pltpu.CompilerParams(use_tc_tiling_on_sc=False),
    )
    def gather_kernel(idx_hbm, table_hbm, out_hbm, flat_v, rows_v, sem, sem2):
        wid = lax.axis_index("s") * _NC + lax.axis_index("c")
        base = wid * rows_per_w

        def body(i, carry):
            r0 = base + i * nb
            pltpu.sync_copy(idx_hbm.at[pl.ds(r0 * hist, n_flat)], flat_v)
            pltpu.async_copy(table_hbm.at[flat_v], rows_v, sem).wait()
            handles = [
                pltpu.async_copy(
                    rows_v.at[pl.ds(r * hist, hist), :],
                    out_hbm.at[r0 + r],
                    sem2,
                )
                for r in range(nb)
            ]
            for h in handles:
                h.wait()
            return carry

        lax.fori_loop(0, n_chunks, body, 0)

    return gather_kernel


def kernel(inputs, table):
    batch, hist = inputs.shape
    n_vocab, emb_dim = table.shape
    idx_flat = inputs.reshape(-1)
    return _make_gather(batch, hist, emb_dim, nb=64)(idx_flat, table)
